# Initial kernel scaffold; baseline (speedup 1.0000x reference)
import jax
import jax.numpy as jnp
from jax.experimental import pallas as pl


def kernel(x, edge_index, params):
    raise NotImplementedError("write your pallas kernel here")


# speedup vs baseline: 290.1766x; 290.1766x over previous
import jax
import jax.numpy as jnp
from jax.experimental import pallas as pl
from jax.experimental.pallas import tpu as pltpu

_TI = 256


def _lrelu(v):
    return jnp.where(v >= 0, v, 0.2 * v)


def _gat_body(x_ref, maskT_ref, wlT_ref, bl_ref, wrT_ref, br_ref, attv_ref,
              atts_ref, bias_ref, out_ref, s_ref, hlT_ref):
    it = pl.program_id(0)
    n = x_ref.shape[0]
    cout = wlT_ref.shape[1]
    x = x_ref[...]
    hp = jax.lax.Precision.HIGHEST
    hl = jnp.dot(x, wlT_ref[...], precision=hp) + bl_ref[...]
    hlT_ref[...] = hl.T
    x_t = x_ref[pl.ds(it * _TI, _TI), :]
    hr_t = jnp.dot(x_t, wrT_ref[...], precision=hp) + br_ref[...]
    hl_t = jnp.dot(x_t, wlT_ref[...], precision=hp) + bl_ref[...]
    attv = attv_ref[...]

    s_ref[...] = jnp.zeros((_TI, n), jnp.float32)
    lane_iota = jax.lax.broadcasted_iota(jnp.int32, (_TI, cout), 1)

    def cbody(c, carry):
        a = atts_ref[0, c]
        col = jnp.sum(jnp.where(lane_iota == c, hr_t, 0.0), axis=1,
                      keepdims=True)
        row = hlT_ref[pl.ds(c, 1), :]
        s_ref[...] += a * _lrelu(col + row)
        return carry

    jax.lax.fori_loop(0, cout, cbody, 0)

    S = s_ref[...]
    mask = maskT_ref[...] > 0
    d = jnp.sum(_lrelu(hr_t + hl_t) * attv, axis=1, keepdims=True)
    mx = jnp.max(jnp.where(mask, S, -jnp.inf), axis=1, keepdims=True)
    mx = jnp.maximum(mx, d)
    P = jnp.where(mask, jnp.exp(S - mx), 0.0)
    p_self = jnp.exp(d - mx)
    denom = jnp.sum(P, axis=1, keepdims=True) + p_self + 1e-16
    num = jnp.dot(P, hl, precision=hp) + p_self * hl_t
    out = num / denom + bias_ref[...]
    out_ref[...] = jnp.maximum(out, 0.0)


def _gat_layer(x, maskT, p):
    n, cin = x.shape
    cout = p["Wl"].shape[0]
    wlT = p["Wl"].T
    wrT = p["Wr"].T
    bl = p["bl"].reshape(1, cout)
    br = p["br"].reshape(1, cout)
    att = p["att"].reshape(1, cout)
    bias = p["bias"].reshape(1, cout)
    return pl.pallas_call(
        _gat_body,
        grid=(n // _TI,),
        in_specs=[
            pl.BlockSpec((n, cin), lambda i: (0, 0)),
            pl.BlockSpec((_TI, n), lambda i: (i, 0)),
            pl.BlockSpec((cin, cout), lambda i: (0, 0)),
            pl.BlockSpec((1, cout), lambda i: (0, 0)),
            pl.BlockSpec((cin, cout), lambda i: (0, 0)),
            pl.BlockSpec((1, cout), lambda i: (0, 0)),
            pl.BlockSpec((1, cout), lambda i: (0, 0)),
            pl.BlockSpec(memory_space=pltpu.SMEM),
            pl.BlockSpec((1, cout), lambda i: (0, 0)),
        ],
        out_specs=pl.BlockSpec((_TI, cout), lambda i: (i, 0)),
        out_shape=jax.ShapeDtypeStruct((n, cout), jnp.float32),
        scratch_shapes=[pltpu.VMEM((_TI, n), jnp.float32),
                        pltpu.VMEM((cout, n), jnp.float32)],
    )(x, maskT, wlT, bl, wrT, br, att, att, bias)


def _recon_body(re_ref, out_ref):
    it = pl.program_id(0)
    n, c = re_ref.shape
    re = re_ref[...]
    re_t = re_ref[pl.ds(it * _TI, _TI), :]
    logits = jnp.dot(re_t, re.T, precision=jax.lax.Precision.HIGHEST)
    out_ref[...] = jax.nn.sigmoid(logits)


def _recon(re):
    n, c = re.shape
    return pl.pallas_call(
        _recon_body,
        grid=(n // _TI,),
        in_specs=[pl.BlockSpec((n, c), lambda i: (0, 0))],
        out_specs=pl.BlockSpec((_TI, n), lambda i: (i, 0)),
        out_shape=jax.ShapeDtypeStruct((n, n), jnp.float32),
    )(re)


def kernel(x, edge_index, params):
    maskT = (edge_index.T != 0).astype(jnp.float32)
    x1 = _gat_layer(x, maskT, params["conv1"])
    z = _gat_layer(x1, maskT, params["conv2"])
    re = _gat_layer(z, maskT, params["edge_dec"])
    recon = _recon(re)
    xd = _gat_layer(z, maskT, params["x_dec1"])
    xr = _gat_layer(xd, maskT, params["x_dec2"])
    return recon, xr, z


# speedup vs baseline: 442.3115x; 1.5243x over previous
import jax
import jax.numpy as jnp
from jax.experimental import pallas as pl
from jax.experimental.pallas import tpu as pltpu

_TI = 256
_CC = 4


def _lrelu(v):
    return jnp.where(v >= 0, v, 0.2 * v)


def _gat_body(x_ref, maskT_ref, wlT_ref, bl_ref, wrT_ref, br_ref, attv_ref,
              atts_ref, bias_ref, out_ref, s_ref, hlsT_ref):
    it = pl.program_id(0)
    n = x_ref.shape[0]
    cout = wlT_ref.shape[1]
    x = x_ref[...]
    hp = jax.lax.Precision.HIGHEST
    attv = attv_ref[...]
    hl = jnp.dot(x, wlT_ref[...], precision=hp) + bl_ref[...]
    hlsT_ref[...] = (hl * attv).T
    x_t = x_ref[pl.ds(it * _TI, _TI), :]
    hr_t = jnp.dot(x_t, wrT_ref[...], precision=hp) + br_ref[...]
    hl_t = jnp.dot(x_t, wlT_ref[...], precision=hp) + bl_ref[...]
    hrs_t = hr_t * attv

    ar = jnp.sum(hrs_t, axis=1, keepdims=True)
    al_row = jnp.sum(hlsT_ref[...], axis=0, keepdims=True)
    s_ref[...] = 0.6 * (ar + al_row)

    lane_iota = jax.lax.broadcasted_iota(jnp.int32, (_TI, cout), 1)

    def cbody(k, carry):
        acc = None
        for u in range(_CC):
            c = k * _CC + u
            a = atts_ref[0, c]
            g = jnp.where(a >= 0, jnp.float32(0.4), jnp.float32(-0.4))
            col = jnp.sum(jnp.where(lane_iota == c, hrs_t, 0.0), axis=1,
                          keepdims=True)
            row = hlsT_ref[pl.ds(c, 1), :]
            term = g * jnp.abs(col + row)
            acc = term if acc is None else acc + term
        s_ref[...] += acc
        return carry

    jax.lax.fori_loop(0, cout // _CC, cbody, 0)

    S = s_ref[...]
    mask = maskT_ref[...] > 0
    d = jnp.sum(_lrelu(hr_t + hl_t) * attv, axis=1, keepdims=True)
    mx = jnp.max(jnp.where(mask, S, -jnp.inf), axis=1, keepdims=True)
    mx = jnp.maximum(mx, d)
    P = jnp.where(mask, jnp.exp(S - mx), 0.0)
    p_self = jnp.exp(d - mx)
    denom = jnp.sum(P, axis=1, keepdims=True) + p_self + 1e-16
    num = jnp.dot(P, hl, precision=hp) + p_self * hl_t
    out = num / denom + bias_ref[...]
    out_ref[...] = jnp.maximum(out, 0.0)


def _gat_layer(x, maskT, p):
    n, cin = x.shape
    cout = p["Wl"].shape[0]
    wlT = p["Wl"].T
    wrT = p["Wr"].T
    bl = p["bl"].reshape(1, cout)
    br = p["br"].reshape(1, cout)
    att = p["att"].reshape(1, cout)
    bias = p["bias"].reshape(1, cout)
    return pl.pallas_call(
        _gat_body,
        grid=(n // _TI,),
        in_specs=[
            pl.BlockSpec((n, cin), lambda i: (0, 0)),
            pl.BlockSpec((_TI, n), lambda i: (i, 0)),
            pl.BlockSpec((cin, cout), lambda i: (0, 0)),
            pl.BlockSpec((1, cout), lambda i: (0, 0)),
            pl.BlockSpec((cin, cout), lambda i: (0, 0)),
            pl.BlockSpec((1, cout), lambda i: (0, 0)),
            pl.BlockSpec((1, cout), lambda i: (0, 0)),
            pl.BlockSpec(memory_space=pltpu.SMEM),
            pl.BlockSpec((1, cout), lambda i: (0, 0)),
        ],
        out_specs=pl.BlockSpec((_TI, cout), lambda i: (i, 0)),
        out_shape=jax.ShapeDtypeStruct((n, cout), jnp.float32),
        scratch_shapes=[pltpu.VMEM((_TI, n), jnp.float32),
                        pltpu.VMEM((cout, n), jnp.float32)],
        compiler_params=pltpu.CompilerParams(
            dimension_semantics=("parallel",)),
    )(x, maskT, wlT, bl, wrT, br, att, att, bias)


def _recon_body(re_ref, out_ref):
    it = pl.program_id(0)
    n, c = re_ref.shape
    re = re_ref[...]
    re_t = re_ref[pl.ds(it * _TI, _TI), :]
    logits = jnp.dot(re_t, re.T, precision=jax.lax.Precision.HIGHEST)
    out_ref[...] = jax.nn.sigmoid(logits)


def _recon(re):
    n, c = re.shape
    return pl.pallas_call(
        _recon_body,
        grid=(n // _TI,),
        in_specs=[pl.BlockSpec((n, c), lambda i: (0, 0))],
        out_specs=pl.BlockSpec((_TI, n), lambda i: (i, 0)),
        out_shape=jax.ShapeDtypeStruct((n, n), jnp.float32),
        compiler_params=pltpu.CompilerParams(
            dimension_semantics=("parallel",)),
    )(re)


def kernel(x, edge_index, params):
    maskT = (edge_index.T != 0).astype(jnp.float32)
    x1 = _gat_layer(x, maskT, params["conv1"])
    z = _gat_layer(x1, maskT, params["conv2"])
    re = _gat_layer(z, maskT, params["edge_dec"])
    recon = _recon(re)
    xd = _gat_layer(z, maskT, params["x_dec1"])
    xr = _gat_layer(xd, maskT, params["x_dec2"])
    return recon, xr, z
